# baseline (speedup 1.0000x reference)
import jax
import jax.numpy as jnp
from jax import lax
from jax.experimental import pallas as pl
from jax.experimental.pallas import tpu as pltpu
from jax.experimental.pallas import tpu_sc as plsc

N = 10000
E = 320000
D = 128

NC = 2
NS = 16
NW = NC * NS
K = 80
EP = E // NW
CAP_G = 128
CAP = 10368
HALF = 5120
AR = 5376
RPT = AR // NS
ZR = 24


def _matmul_body(h_ref, w_ref, b_ref, o_ref):
    o_ref[...] = (
        jnp.dot(h_ref[...], w_ref[...], preferred_element_type=jnp.float32)
        + b_ref[...]
    )


def _combine_body(p_ref, o_ref):
    o_ref[...] = jnp.maximum(p_ref[0, 0] + p_ref[1, 0], 0.0)


def _sc_body(hw, src, dst, av, out, csrc, cdst1, cav, cdst2, rows_a, rows_b,
             zbuf, acc, gsa, gsb, ssa, ssb):
    c = lax.axis_index("c")
    s = lax.axis_index("s")
    wid = c * NS + s

    def _zero_row(i, _):
        for j in range(D // 16):
            zbuf[i, pl.ds(j * 16, 16)] = jnp.zeros((16,), jnp.float32)
        return 0

    lax.fori_loop(0, ZR, _zero_row, 0)

    dump = HALF + s * 16 + lax.iota(jnp.int32, 16)
    zeros16i = jnp.zeros((16,), jnp.int32)

    def _scale_buf(buf, g):
        def _scale(e, _):
            ab = plsc.load_gather(cav, [jnp.full((16,), g * K + e, jnp.int32)])
            for j in range(D // 16):
                sl = pl.ds(j * 16, 16)
                buf[e, sl] = buf[e, sl] * ab
            return 0

        lax.fori_loop(0, K, _scale, 0)

    for p in range(2):
        pltpu.sync_copy(src.at[wid], csrc)
        pltpu.sync_copy(dst.at[wid], cdst1)
        pltpu.sync_copy(av.at[wid], cav)

        def _compact(i, off):
            sl = pl.ds(i * 16, 16)
            local = cdst1[sl] - p * HALF
            s16 = csrc[sl]
            a16 = cav[sl]
            keep = (local >= 0) & (local < HALF)
            osl = pl.ds(off, 16)
            plsc.store_compressed(cdst1.at[osl], local, mask=keep)
            plsc.store_compressed(csrc.at[osl], s16, mask=keep)
            plsc.store_compressed(cav.at[osl], a16, mask=keep)
            return off + jnp.sum(keep.astype(jnp.int32))

        cnt = lax.fori_loop(0, CAP // 16, _compact, jnp.int32(0))
        npair = jnp.maximum((cnt + 2 * K - 1) // (2 * K), 1)

        for f in range(21):
            fsl = pl.ds(cnt + f * 16, 16)
            csrc[fsl] = zeros16i
            cdst1[fsl] = dump

        def _c2d(r, _):
            for c5 in range(K // 16):
                cdst2[r, pl.ds(c5 * 16, 16)] = cdst1[pl.ds(r * K + c5 * 16, 16)]
            return 0

        lax.fori_loop(0, CAP_G, _c2d, 0)

        for r in range(RPT // ZR):
            pltpu.sync_copy(zbuf, acc.at[pl.ds(s * RPT + r * ZR, ZR)])
        plsc.subcore_barrier()

        pltpu.async_copy(hw.at[csrc.at[pl.ds(0, K)]], rows_a, gsa)

        def _pair(t, _):
            g0 = 2 * t
            g1 = 2 * t + 1

            @pl.when(t > 0)
            def _():
                pltpu.make_async_copy(rows_b, acc.at[cdst2.at[g1]], ssb).wait()

            pltpu.async_copy(hw.at[csrc.at[pl.ds(g1 * K, K)]], rows_b, gsb)

            pltpu.make_async_copy(hw.at[csrc.at[pl.ds(g0 * K, K)]], rows_a,
                                  gsa).wait()
            _scale_buf(rows_a, g0)
            pltpu.async_copy(rows_a, acc.at[cdst2.at[g0]], ssa, add=True)

            pltpu.make_async_copy(hw.at[csrc.at[pl.ds(g1 * K, K)]], rows_b,
                                  gsb).wait()
            _scale_buf(rows_b, g1)
            pltpu.async_copy(rows_b, acc.at[cdst2.at[g1]], ssb, add=True)

            pltpu.make_async_copy(rows_a, acc.at[cdst2.at[g0]], ssa).wait()

            @pl.when(t < npair - 1)
            def _():
                pltpu.async_copy(hw.at[csrc.at[pl.ds((g0 + 2) * K, K)]],
                                 rows_a, gsa)

            return 0

        lax.fori_loop(0, npair, _pair, 0)

        pltpu.make_async_copy(rows_b, acc.at[cdst2.at[1]], ssb).wait()
        plsc.subcore_barrier()

        pltpu.sync_copy(
            acc.at[pl.ds(s * RPT, RPT)],
            out.at[c, p, pl.ds(s * RPT, RPT)],
        )
        plsc.subcore_barrier()


def kernel(H, edge_index, A_vals, W, b):
    hw = pl.pallas_call(
        _matmul_body,
        grid=(10,),
        in_specs=[
            pl.BlockSpec((N // 10, D), lambda i: (i, 0)),
            pl.BlockSpec((D, D), lambda i: (0, 0)),
            pl.BlockSpec((1, D), lambda i: (0, 0)),
        ],
        out_specs=pl.BlockSpec((N // 10, D), lambda i: (i, 0)),
        out_shape=jax.ShapeDtypeStruct((N, D), jnp.float32),
    )(H, W, b.reshape(1, D))

    pad = ((0, 0), (0, CAP - EP))
    src2 = jnp.pad(edge_index[0].reshape(NW, EP), pad)
    dst2 = jnp.pad(edge_index[1].reshape(NW, EP), pad,
                   constant_values=2 * HALF)
    av2 = jnp.pad(A_vals.reshape(NW, EP), pad)

    mesh = plsc.VectorSubcoreMesh(
        core_axis_name="c", subcore_axis_name="s", num_cores=NC, num_subcores=NS
    )
    scatter = pl.kernel(
        _sc_body,
        out_type=jax.ShapeDtypeStruct((NC, 2, AR, D), jnp.float32),
        mesh=mesh,
        compiler_params=pltpu.CompilerParams(needs_layout_passes=False),
        scratch_types=[
            pltpu.VMEM((CAP,), jnp.int32),
            pltpu.VMEM((CAP,), jnp.int32),
            pltpu.VMEM((CAP,), jnp.float32),
            pltpu.VMEM((CAP_G, K), jnp.int32),
            pltpu.VMEM((K, D), jnp.float32),
            pltpu.VMEM((K, D), jnp.float32),
            pltpu.VMEM((ZR, D), jnp.float32),
            pltpu.VMEM_SHARED((AR, D), jnp.float32),
            pltpu.SemaphoreType.DMA,
            pltpu.SemaphoreType.DMA,
            pltpu.SemaphoreType.DMA,
            pltpu.SemaphoreType.DMA,
        ],
    )
    partials = scatter(hw, src2, dst2, av2)

    out = pl.pallas_call(
        _combine_body,
        grid=(2, 10),
        in_specs=[
            pl.BlockSpec((NC, 1, 512, D), lambda q, r: (0, q, r, 0)),
        ],
        out_specs=pl.BlockSpec((512, D), lambda q, r: (q * 10 + r, 0)),
        out_shape=jax.ShapeDtypeStruct((N, D), jnp.float32),
    )(partials)
    return out


# speedup vs baseline: 1.3507x; 1.3507x over previous
import jax
import jax.numpy as jnp
from jax import lax
from jax.experimental import pallas as pl
from jax.experimental.pallas import tpu as pltpu
from jax.experimental.pallas import tpu_sc as plsc

N = 10000
E = 320000
D = 128

NC = 2
NS = 16
NW = NC * NS
K = 80
EP = E // NW
G = EP // K
HALF = 5120
AR = 5376
RPT = AR // NS
ZR = 24


def _matmul_body(h_ref, w_ref, b_ref, o_ref):
    o_ref[...] = (
        jnp.dot(h_ref[...], w_ref[...], preferred_element_type=jnp.float32)
        + b_ref[...]
    )


def _combine_body(p_ref, o_ref):
    o_ref[...] = jnp.maximum(p_ref[0, 0] + p_ref[1, 0], 0.0)


def _sc_body(hw, src, dst, av, out, src_v, dst_v, dstp_v, av_v, rows_a,
             rows_b, zbuf, acc, gsa, gsb, ssa, ssb):
    c = lax.axis_index("c")
    s = lax.axis_index("s")
    wid = c * NS + s

    def _zero_row(i, _):
        for j in range(D // 16):
            zbuf[i, pl.ds(j * 16, 16)] = jnp.zeros((16,), jnp.float32)
        return 0

    lax.fori_loop(0, ZR, _zero_row, 0)

    pltpu.sync_copy(src.at[wid], src_v)
    pltpu.sync_copy(dst.at[wid], dst_v)
    pltpu.sync_copy(av.at[wid], av_v)

    dump = HALF + s * 16 + lax.iota(jnp.int32, 16)

    def _scale_buf(buf, g):
        @plsc.parallel_loop(0, K, step=1, unroll=4)
        def _scale(e):
            ab = plsc.load_gather(av_v, [jnp.full((16,), g * K + e, jnp.int32)])
            for j in range(D // 16):
                sl = pl.ds(j * 16, 16)
                buf[e, sl] = buf[e, sl] * ab

    for p in range(2):
        def _remap(r, _):
            for c5 in range(K // 16):
                sl = pl.ds(c5 * 16, 16)
                d16 = dst_v[r, sl]
                local = d16 - p * HALF
                oob = (local < 0) | (local >= HALF)
                dstp_v[r, sl] = jnp.where(oob, dump, local)
            return 0

        lax.fori_loop(0, G, _remap, 0)

        for r in range(RPT // ZR):
            pltpu.sync_copy(zbuf, acc.at[pl.ds(s * RPT + r * ZR, ZR)])
        plsc.subcore_barrier()

        pltpu.async_copy(hw.at[src_v.at[0]], rows_a, gsa)

        def _pair(t, _):
            g0 = 2 * t
            g1 = 2 * t + 1

            @pl.when(t > 0)
            def _():
                pltpu.make_async_copy(rows_b, acc.at[dstp_v.at[g1]], ssb).wait()

            pltpu.async_copy(hw.at[src_v.at[g1]], rows_b, gsb)

            pltpu.make_async_copy(hw.at[src_v.at[g0]], rows_a, gsa).wait()
            _scale_buf(rows_a, g0)
            pltpu.async_copy(rows_a, acc.at[dstp_v.at[g0]], ssa, add=True)

            pltpu.make_async_copy(hw.at[src_v.at[g1]], rows_b, gsb).wait()
            _scale_buf(rows_b, g1)
            pltpu.async_copy(rows_b, acc.at[dstp_v.at[g1]], ssb, add=True)

            pltpu.make_async_copy(rows_a, acc.at[dstp_v.at[g0]], ssa).wait()
            pltpu.async_copy(hw.at[src_v.at[g0 + 2]], rows_a, gsa)
            return 0

        lax.fori_loop(0, G // 2, _pair, 0)

        pltpu.make_async_copy(rows_b, acc.at[dstp_v.at[G - 2]], ssb).wait()
        pltpu.make_async_copy(hw.at[src_v.at[G - 1]], rows_a, gsa).wait()
        _scale_buf(rows_a, G - 1)
        pltpu.sync_copy(rows_a, acc.at[dstp_v.at[G - 1]], add=True)
        plsc.subcore_barrier()

        pltpu.sync_copy(
            acc.at[pl.ds(s * RPT, RPT)],
            out.at[c, p, pl.ds(s * RPT, RPT)],
        )
        plsc.subcore_barrier()


def kernel(H, edge_index, A_vals, W, b):
    hw = pl.pallas_call(
        _matmul_body,
        grid=(10,),
        in_specs=[
            pl.BlockSpec((N // 10, D), lambda i: (i, 0)),
            pl.BlockSpec((D, D), lambda i: (0, 0)),
            pl.BlockSpec((1, D), lambda i: (0, 0)),
        ],
        out_specs=pl.BlockSpec((N // 10, D), lambda i: (i, 0)),
        out_shape=jax.ShapeDtypeStruct((N, D), jnp.float32),
    )(H, W, b.reshape(1, D))

    src2 = edge_index[0].reshape(NW, G, K)
    dst2 = edge_index[1].reshape(NW, G, K)
    av2 = A_vals.reshape(NW, G * K)

    mesh = plsc.VectorSubcoreMesh(
        core_axis_name="c", subcore_axis_name="s", num_cores=NC, num_subcores=NS
    )
    scatter = pl.kernel(
        _sc_body,
        out_type=jax.ShapeDtypeStruct((NC, 2, AR, D), jnp.float32),
        mesh=mesh,
        compiler_params=pltpu.CompilerParams(needs_layout_passes=False),
        scratch_types=[
            pltpu.VMEM((G, K), jnp.int32),
            pltpu.VMEM((G, K), jnp.int32),
            pltpu.VMEM((G, K), jnp.int32),
            pltpu.VMEM((G * K,), jnp.float32),
            pltpu.VMEM((K, D), jnp.float32),
            pltpu.VMEM((K, D), jnp.float32),
            pltpu.VMEM((ZR, D), jnp.float32),
            pltpu.VMEM_SHARED((AR, D), jnp.float32),
            pltpu.SemaphoreType.DMA,
            pltpu.SemaphoreType.DMA,
            pltpu.SemaphoreType.DMA,
            pltpu.SemaphoreType.DMA,
        ],
    )
    partials = scatter(hw, src2, dst2, av2)

    out = pl.pallas_call(
        _combine_body,
        grid=(2, 10),
        in_specs=[
            pl.BlockSpec((NC, 1, 512, D), lambda q, r: (0, q, r, 0)),
        ],
        out_specs=pl.BlockSpec((512, D), lambda q, r: (q * 10 + r, 0)),
        out_shape=jax.ShapeDtypeStruct((N, D), jnp.float32),
    )(partials)
    return out
